# baseline (speedup 1.0000x reference)
import functools

import jax
import jax.numpy as jnp
from jax import lax
from jax.experimental import pallas as pl
from jax.experimental.pallas import tpu as pltpu
from jax.experimental.pallas import tpu_sc as plsc

_MARGIN = 0.2
_NC = 2
_NS = 16
_NW = _NC * _NS
_L = 16
_D = 128
_G = 128
_T = 131072
_STEPS = _T // (_NW * _G)
_TRI_UNROLL = 2


def _make_tri_call():
    mesh = plsc.VectorSubcoreMesh(
        core_axis_name="c", subcore_axis_name="s",
        num_cores=_NC, num_subcores=_NS)

    @functools.partial(
        pl.kernel,
        out_type=jax.ShapeDtypeStruct((_NW, _L), jnp.float32),
        mesh=mesh,
        compiler_params=pltpu.CompilerParams(needs_layout_passes=False),
        scratch_types=[
            pltpu.VMEM((_STEPS, 3, _G), jnp.int32),
            pltpu.VMEM((_G, _D), jnp.float32),
            pltpu.VMEM((_G, _D), jnp.float32),
            pltpu.VMEM((_G, _D), jnp.float32),
            pltpu.VMEM((_G, _D), jnp.float32),
            pltpu.VMEM((_G, _D), jnp.float32),
            pltpu.VMEM((_G, _D), jnp.float32),
            pltpu.VMEM((_L,), jnp.float32),
            pltpu.SemaphoreType.DMA,
            pltpu.SemaphoreType.DMA,
        ],
    )
    def tri_kernel(batch_hbm, idx_hbm, out_hbm,
                   idx_all, a0, p0, n0, a1, p1, n1, out_v, sem0, sem1):
        wid = lax.axis_index("s") * _NC + lax.axis_index("c")
        bufs = ((a0, p0, n0), (a1, p1, n1))
        sems = (sem0, sem1)

        pltpu.sync_copy(idx_hbm.at[wid], idx_all)

        def start(step, b):
            for j in range(3):
                pltpu.async_copy(batch_hbm.at[idx_all.at[step, j]],
                                 bufs[b][j], sems[b])

        def wait(step, b):
            for j in range(3):
                pltpu.make_async_copy(batch_hbm.at[idx_all.at[step, j]],
                                      bufs[b][j], sems[b]).wait()

        def compute(b, acc):
            a_ref, p_ref, n_ref = bufs[b]
            return acc + jnp.sum(a_ref[0, 0:_L] + p_ref[0, 0:_L]
                                 + n_ref[0, 0:_L])

            def one_triplet(i):
                s0 = jnp.zeros((_L,), jnp.float32)
                s1 = jnp.zeros((_L,), jnp.float32)
                for c in range(_D // _L):
                    sl = pl.ds(c * _L, _L)
                    av = a_ref[i, sl]
                    pv = p_ref[i, sl]
                    nv = n_ref[i, sl]
                    dp = av - pv
                    dn = av - nv
                    t = dp * dp - dn * dn
                    if c % 2 == 0:
                        s0 = s0 + t
                    else:
                        s1 = s1 + t
                tot = jnp.sum(s0 + s1)
                return jnp.maximum(tot + _MARGIN, 0.0)

            def body(it, a):
                for k in range(_TRI_UNROLL):
                    a = a + one_triplet(it * _TRI_UNROLL + k)
                return a

            return lax.fori_loop(0, _G // _TRI_UNROLL, body, acc)

        start(0, 0)
        acc0 = jnp.float32(0.0)

        def outer(s2, acc):
            step = 2 * s2
            start(step + 1, 1)
            wait(step, 0)
            acc = compute(0, acc)

            @pl.when(s2 + 1 < _STEPS // 2)
            def _():
                start(step + 2, 0)

            wait(step + 1, 1)
            acc = compute(1, acc)
            return acc

        acc0 = lax.fori_loop(0, _STEPS // 2, outer, acc0)
        out_v[...] = jnp.where(lax.iota(jnp.int32, _L) == 0, acc0, 0.0)
        pltpu.sync_copy(out_v, out_hbm.at[wid])

    return tri_kernel


_tri_call = _make_tri_call()


def kernel(batch, labels, triplets, step):
    del labels, step
    idx_arr = triplets.reshape(_NW, _STEPS, _G, 3).transpose(0, 1, 3, 2)
    partials = _tri_call(batch, idx_arr)
    return jnp.sum(partials) / jnp.float32(_T)


# speedup vs baseline: 1.0260x; 1.0260x over previous
import functools

import jax
import jax.numpy as jnp
from jax import lax
from jax.experimental import pallas as pl
from jax.experimental.pallas import tpu as pltpu
from jax.experimental.pallas import tpu_sc as plsc

_MARGIN = 0.2
_NC = 2
_NS = 16
_NW = _NC * _NS
_L = 16
_D = 128
_G = 128
_T = 131072
_STEPS = _T // (_NW * _G)
_TRI_UNROLL = 2


def _make_tri_call():
    mesh = plsc.VectorSubcoreMesh(
        core_axis_name="c", subcore_axis_name="s",
        num_cores=_NC, num_subcores=_NS)

    @functools.partial(
        pl.kernel,
        out_type=jax.ShapeDtypeStruct((_NW, _L), jnp.float32),
        mesh=mesh,
        compiler_params=pltpu.CompilerParams(needs_layout_passes=False),
        scratch_types=[
            pltpu.VMEM((_STEPS, 3, _G), jnp.int32),
            pltpu.VMEM((_G, _D), jnp.float32),
            pltpu.VMEM((_G, _D), jnp.float32),
            pltpu.VMEM((_G, _D), jnp.float32),
            pltpu.VMEM((_G, _D), jnp.float32),
            pltpu.VMEM((_G, _D), jnp.float32),
            pltpu.VMEM((_G, _D), jnp.float32),
            pltpu.VMEM((_L,), jnp.float32),
            pltpu.SemaphoreType.DMA,
            pltpu.SemaphoreType.DMA,
        ],
    )
    def tri_kernel(batch_hbm, idx_hbm, out_hbm,
                   idx_all, a0, p0, n0, a1, p1, n1, out_v, sem0, sem1):
        wid = lax.axis_index("s") * _NC + lax.axis_index("c")
        bufs = ((a0, p0, n0), (a1, p1, n1))
        sems = (sem0, sem1)

        pltpu.sync_copy(idx_hbm.at[wid], idx_all)

        def start(step, b):
            for j in range(3):
                pltpu.async_copy(batch_hbm.at[idx_all.at[step, j]],
                                 bufs[b][j], sems[b])

        def wait(step, b):
            for j in range(3):
                pltpu.make_async_copy(batch_hbm.at[idx_all.at[step, j]],
                                      bufs[b][j], sems[b]).wait()

        def compute(b, acc):
            a_ref, p_ref, n_ref = bufs[b]

            def one_triplet(i):
                s0 = jnp.zeros((_L,), jnp.float32)
                s1 = jnp.zeros((_L,), jnp.float32)
                for c in range(_D // _L):
                    sl = pl.ds(c * _L, _L)
                    av = a_ref[i, sl]
                    pv = p_ref[i, sl]
                    nv = n_ref[i, sl]
                    dp = av - pv
                    dn = av - nv
                    t = dp * dp - dn * dn
                    if c % 2 == 0:
                        s0 = s0 + t
                    else:
                        s1 = s1 + t
                tot = jnp.sum(s0 + s1)
                return jnp.maximum(tot + _MARGIN, 0.0)

            def body(it, a):
                for k in range(_TRI_UNROLL):
                    a = a + one_triplet(it * _TRI_UNROLL + k)
                return a

            return lax.fori_loop(0, _G // _TRI_UNROLL, body, acc)

        start(0, 0)
        acc0 = jnp.float32(0.0)

        def outer(s2, acc):
            step = 2 * s2
            start(step + 1, 1)
            wait(step, 0)
            acc = compute(0, acc)

            @pl.when(s2 + 1 < _STEPS // 2)
            def _():
                start(step + 2, 0)

            wait(step + 1, 1)
            acc = compute(1, acc)
            return acc

        acc0 = lax.fori_loop(0, _STEPS // 2, outer, acc0)
        out_v[...] = jnp.where(lax.iota(jnp.int32, _L) == 0, acc0, 0.0)
        pltpu.sync_copy(out_v, out_hbm.at[wid])

    return tri_kernel


_tri_call = _make_tri_call()


def kernel(batch, labels, triplets, step):
    del labels, step
    idx_arr = triplets.reshape(_NW, _STEPS, _G, 3).transpose(0, 1, 3, 2)
    partials = _tri_call(batch, idx_arr)
    return jnp.sum(partials) / jnp.float32(_T)


# speedup vs baseline: 1.0464x; 1.0200x over previous
import functools

import jax
import jax.numpy as jnp
from jax import lax
from jax.experimental import pallas as pl
from jax.experimental.pallas import tpu as pltpu
from jax.experimental.pallas import tpu_sc as plsc

_MARGIN = 0.2
_NC = 2
_NS = 16
_NW = _NC * _NS
_L = 16
_D = 128
_G = 128
_T = 131072
_STEPS = _T // (_NW * _G)
_TRI_UNROLL = 2


def _make_tri_call():
    mesh = plsc.VectorSubcoreMesh(
        core_axis_name="c", subcore_axis_name="s",
        num_cores=_NC, num_subcores=_NS)

    @functools.partial(
        pl.kernel,
        out_type=jax.ShapeDtypeStruct((_NW, _L), jnp.float32),
        mesh=mesh,
        compiler_params=pltpu.CompilerParams(needs_layout_passes=False),
        scratch_types=[
            pltpu.VMEM((_STEPS, 3, _G), jnp.int32),
            pltpu.VMEM((_G, _D), jnp.float32),
            pltpu.VMEM((_G, _D), jnp.float32),
            pltpu.VMEM((_G, _D), jnp.float32),
            pltpu.VMEM((_G, _D), jnp.float32),
            pltpu.VMEM((_G, _D), jnp.float32),
            pltpu.VMEM((_G, _D), jnp.float32),
            pltpu.VMEM((_L,), jnp.float32),
            pltpu.SemaphoreType.DMA,
            pltpu.SemaphoreType.DMA,
        ],
    )
    def tri_kernel(batch_hbm, idx_hbm, out_hbm,
                   idx_all, a0, p0, n0, a1, p1, n1, out_v, sem0, sem1):
        wid = lax.axis_index("s") * _NC + lax.axis_index("c")
        bufs = ((a0, p0, n0), (a1, p1, n1))
        sems = (sem0, sem1)

        pltpu.sync_copy(idx_hbm.at[wid, 0], idx_all.at[0])

        def start(step, b):
            for j in range(3):
                pltpu.async_copy(batch_hbm.at[idx_all.at[step, j]],
                                 bufs[b][j], sems[b])

        def wait(step, b):
            for j in range(3):
                pltpu.make_async_copy(batch_hbm.at[idx_all.at[step, j]],
                                      bufs[b][j], sems[b]).wait()

        def compute(b, acc):
            a_ref, p_ref, n_ref = bufs[b]

            def one_triplet(i):
                s0 = jnp.zeros((_L,), jnp.float32)
                s1 = jnp.zeros((_L,), jnp.float32)
                for c in range(_D // _L):
                    sl = pl.ds(c * _L, _L)
                    av = a_ref[i, sl]
                    pv = p_ref[i, sl]
                    nv = n_ref[i, sl]
                    dp = av - pv
                    dn = av - nv
                    t = dp * dp - dn * dn
                    if c % 2 == 0:
                        s0 = s0 + t
                    else:
                        s1 = s1 + t
                tot = jnp.sum(s0 + s1)
                return jnp.maximum(tot + _MARGIN, 0.0)

            def body(it, a):
                for k in range(_TRI_UNROLL):
                    a = a + one_triplet(it * _TRI_UNROLL + k)
                return a

            return lax.fori_loop(0, _G // _TRI_UNROLL, body, acc)

        start(0, 0)
        pltpu.sync_copy(idx_hbm.at[wid, pl.ds(1, _STEPS - 1)],
                        idx_all.at[pl.ds(1, _STEPS - 1)])
        acc0 = jnp.float32(0.0)

        def outer(s2, acc):
            step = 2 * s2
            start(step + 1, 1)
            wait(step, 0)
            acc = compute(0, acc)

            @pl.when(s2 + 1 < _STEPS // 2)
            def _():
                start(step + 2, 0)

            wait(step + 1, 1)
            acc = compute(1, acc)
            return acc

        acc0 = lax.fori_loop(0, _STEPS // 2, outer, acc0)
        out_v[...] = jnp.where(lax.iota(jnp.int32, _L) == 0, acc0, 0.0)
        pltpu.sync_copy(out_v, out_hbm.at[wid])

    return tri_kernel


_tri_call = _make_tri_call()


def kernel(batch, labels, triplets, step):
    del labels, step
    idx_arr = triplets.reshape(_NW, _STEPS, _G, 3).transpose(0, 1, 3, 2)
    partials = _tri_call(batch, idx_arr)
    return jnp.sum(partials) / jnp.float32(_T)
